# baseline (speedup 1.0000x reference)
import math

import jax
import jax.numpy as jnp
import numpy as np
from jax.experimental import pallas as pl

_S, _N, _NP = 8, 97, 104
_R = _S * _NP


def _pe8_np():
    pos = np.arange(20, dtype=np.float32)[:, None]
    div = np.exp(np.arange(0, 16, 2, dtype=np.float32) * (-math.log(10000.0) / 16.0))
    pe = np.zeros((20, 16), dtype=np.float32)
    pe[:, 0::2] = np.sin(pos * div)
    pe[:, 1::2] = np.cos(pos * div)
    return pe[:_S]


_PE8 = _pe8_np()

_B_SPECS = [
    ("lin0_b", 4), ("lin1_b", 4), ("lin2_b", 8), ("lin2_w", 8),
    ("gc10_b", 64), ("gc11_b", 32), ("gc20_b", 64), ("gc21_b", 32),
    ("gc30_b", 64), ("gc31_b", 32), ("gcn_ln_g", 16), ("gcn_ln_b", 16),
    ("attn_in_b", 48), ("attn_out_b", 16), ("norm1_g", 16), ("norm1_b", 16),
    ("norm2_g", 16), ("norm2_b", 16), ("enc_norm_g", 16), ("enc_norm_b", 16),
    ("pred_b", 8), ("out0_b", 4), ("out1_b", 1), ("ff2_b", 16), ("ff1_b", 2048),
]
_B_OFF = {}
_o = 0
for _n, _w in _B_SPECS:
    _B_OFF[_n] = (_o, _w)
    _o += _w
_B_LANES = _o


def _mmh_t(x, w):
    return jax.lax.dot_general(
        x, w, (((1,), (1,)), ((), ())),
        precision=jax.lax.Precision.HIGHEST,
        preferred_element_type=jnp.float32,
    )


def _mmh(x, w):
    return jax.lax.dot_general(
        x, w, (((1,), (0,)), ((), ())),
        precision=jax.lax.Precision.HIGHEST,
        preferred_element_type=jnp.float32,
    )


_BF = jnp.bfloat16


def _mmd_t(x, w):
    return jax.lax.dot_general(
        x.astype(_BF), w.astype(_BF), (((1,), (1,)), ((), ())),
        preferred_element_type=jnp.float32,
    )


def _mmd(x, w):
    return jax.lax.dot_general(
        x.astype(_BF), w.astype(_BF), (((1,), (0,)), ((), ())),
        preferred_element_type=jnp.float32,
    )


def _bf(x):
    return x.astype(_BF).astype(jnp.float32)


def _ln(x, g, b, eps):
    m = jnp.mean(x, axis=-1, keepdims=True)
    v = jnp.mean((x - m) * (x - m), axis=-1, keepdims=True)
    return (x - m) / jnp.sqrt(v + eps) * g + b


def _sel(rows, cols, shift):
    r = jax.lax.broadcasted_iota(jnp.int32, (rows, cols), 0)
    c = jax.lax.broadcasted_iota(jnp.int32, (rows, cols), 1)
    return (c == r + shift).astype(jnp.float32)


def _blk(x, i):
    return x[i * _NP : (i + 1) * _NP, :]


def _fused_body(
    feat, week, stamp, a0, a1, a2, pe, bpack,
    emb1, emb2, lin0_w, lin1_w,
    gc10_w, gc11_w, gc20_w, gc21_w, gc30_w, gc31_w,
    fw0, fw1, fw2, attn_in_w, attn_out_w, ff1_w, ff2_w,
    pred_w, out0_w, out1_w,
    r1_ref, r2_ref, r2l_ref,
):
    f32 = jnp.float32

    def B(name):
        off, w = _B_OFF[name]
        return bpack[:, off : off + w]

    A0 = jnp.pad(a0[:, :], ((0, _NP - _N), (0, _NP - _N)))
    A1 = jnp.pad(a1[:, :], ((0, _NP - _N), (0, _NP - _N)))
    A2 = jnp.pad(a2[:, :], ((0, _NP - _N), (0, _NP - _N)))

    featc = jnp.pad(jnp.transpose(feat[:, :]), ((0, _NP - _N), (0, 0)))
    weekc = jnp.pad(jnp.transpose(week[:, :]), ((0, _NP - _N), (0, 0)))
    stampc = jnp.pad(jnp.transpose(stamp[:, :]), ((0, _NP - _N), (0, 0)))
    fcol = jnp.concatenate([featc[:, i : i + 1] for i in range(_S)], axis=0)
    oh_w = jnp.concatenate(
        [
            (weekc[:, i : i + 1] == jax.lax.broadcasted_iota(jnp.int32, (_NP, 8), 1)).astype(f32)
            for i in range(_S)
        ],
        axis=0,
    )
    oh_s = jnp.concatenate(
        [
            (stampc[:, i : i + 1] == jax.lax.broadcasted_iota(jnp.int32, (_NP, 5), 1)).astype(f32)
            for i in range(_S)
        ],
        axis=0,
    )

    g1 = _mmh(oh_w, emb1[:, :])
    g2 = _mmh(oh_s, emb2[:, :])
    x1e = _mmd_t(g1, lin0_w[:, :]) + B("lin0_b")
    x2e = _mmd_t(g2, lin1_w[:, :]) + B("lin1_b")
    x3e = fcol * B("lin2_w") + B("lin2_b")

    X = (
        _mmh(x3e, _sel(8, 16, 0))
        + _mmh(x2e, _sel(4, 16, 8))
        + _mmh(x1e, _sel(4, 16, 12))
    )

    def gcn_branch(A, w0, b0, w1, b1):
        U = _mmd(X, w0[:, :])
        V = jnp.concatenate([_mmd(A, _blk(U, i)) for i in range(_S)], axis=0)
        H = jnp.maximum(V + b0, 0.0)
        Wd = _mmd(H, w1[:, :])
        Z = jnp.concatenate([_mmd(A, _blk(Wd, i)) for i in range(_S)], axis=0)
        return Z + b1

    z0 = gcn_branch(A0, gc10_w, B("gc10_b"), gc11_w, B("gc11_b"))
    z1 = gcn_branch(A1, gc20_w, B("gc20_b"), gc21_w, B("gc21_b"))
    z2 = gcn_branch(A2, gc30_w, B("gc30_b"), gc31_w, B("gc31_b"))
    xo = _mmd(z0, fw0[:, :]) + _mmd(z1, fw1[:, :]) + _mmd(z2, fw2[:, :])
    xg = _ln(xo + X, B("gcn_ln_g"), B("gcn_ln_b"), 1e-6)

    step_oh = (
        jax.lax.broadcasted_iota(jnp.int32, (_R, 8), 0) // _NP
        == jax.lax.broadcasted_iota(jnp.int32, (_R, 8), 1)
    ).astype(f32)
    src = xg + _mmh(step_oh, pe[:, :])

    wq = attn_in_w[0:16, :]
    wk = attn_in_w[16:32, :]
    wv = attn_in_w[32:48, :]
    attn_in_b = B("attn_in_b")
    bq = _mmh(attn_in_b, _sel(16, 48, 0).T)
    bk = _mmh(attn_in_b, _sel(16, 48, 16).T)
    bv = _mmh(attn_in_b, _sel(16, 48, 32).T)
    q_all = _bf(_mmd_t(src, wq) + bq)
    k_all = _bf(_mmd_t(src, wk) + bk)
    v_all = _bf(_mmd_t(src, wv) + bv)

    G = (
        jax.lax.broadcasted_iota(jnp.int32, (16, 4), 0) // 4
        == jax.lax.broadcasted_iota(jnp.int32, (16, 4), 1)
    ).astype(f32)

    def tile_steps(x):
        return jnp.concatenate([x] * _S, axis=0)

    scores = []
    for j in range(_S):
        kt = tile_steps(_blk(k_all, j))
        scores.append(_mmh(q_all * kt, G) * 0.5)
    m = scores[0]
    for j in range(1, _S):
        m = jnp.maximum(m, scores[j])
    exps = [jnp.exp(s - m) for s in scores]
    den = exps[0]
    for j in range(1, _S):
        den = den + exps[j]
    ao = jnp.zeros((_R, 16), f32)
    for j in range(_S):
        vt = tile_steps(_blk(v_all, j))
        ao = ao + _mmh(_bf(exps[j] / den), G.T) * vt

    ao = _mmd_t(ao, attn_out_w[:, :]) + B("attn_out_b")
    x1 = _ln(src + ao, B("norm1_g"), B("norm1_b"), 1e-5)
    h = jnp.maximum(_mmd_t(x1, ff1_w[:, :]) + B("ff1_b"), 0.0)
    y = _mmd_t(h, ff2_w[:, :]) + B("ff2_b")
    x2 = _ln(x1 + y, B("norm2_g"), B("norm2_b"), 1e-5)
    enc = _ln(x2, B("enc_norm_g"), B("enc_norm_b"), 1e-6)

    r1 = _mmd_t(enc, pred_w[:, :]) + B("pred_b")
    rb = _mmd_t(r1, out0_w[:, :]) + B("out0_b")
    r2 = jnp.sum(_bf(rb) * _bf(out1_w[:, :]), axis=-1, keepdims=True) + B("out1_b")[0, 0]
    for i in range(_S):
        r1_ref[i] = r1[i * _NP : i * _NP + _N, :]
        r2_ref[i] = r2[i * _NP : i * _NP + _N, :]
    r2l_ref[:, :] = r2[(_S - 1) * _NP : (_S - 1) * _NP + _N, :]


def kernel(feature_tensor, week_tensor, stamptensor, a0, a1, a2, k, params):
    p = params
    del k
    bpack = jnp.concatenate(
        [p[name].reshape(1, -1) for name, _ in _B_SPECS], axis=1
    )
    return tuple(
        pl.pallas_call(
            _fused_body,
            out_shape=[
                jax.ShapeDtypeStruct((_S, _N, 8), jnp.float32),
                jax.ShapeDtypeStruct((_S, _N, 1), jnp.float32),
                jax.ShapeDtypeStruct((_N, 1), jnp.float32),
            ],
        )(
            feature_tensor, week_tensor, stamptensor, a0, a1, a2,
            jnp.asarray(_PE8), bpack,
            p["emb1"], p["emb2"], p["lin0_w"], p["lin1_w"],
            p["gc10_w"], p["gc11_w"], p["gc20_w"], p["gc21_w"],
            p["gc30_w"], p["gc31_w"],
            p["fw0"], p["fw1"], p["fw2"],
            p["attn_in_w"], p["attn_out_w"],
            p["ff1_w"], p["ff2_w"],
            p["pred_w"], p["out0_w"], p["out1_w"],
        )
    )


# speedup vs baseline: 1.1865x; 1.1865x over previous
import math

import jax
import jax.numpy as jnp
import numpy as np
from jax.experimental import pallas as pl

_S, _N, _NP = 8, 97, 104
_R = _S * _NP


def _pe8_np():
    pos = np.arange(20, dtype=np.float32)[:, None]
    div = np.exp(np.arange(0, 16, 2, dtype=np.float32) * (-math.log(10000.0) / 16.0))
    pe = np.zeros((20, 16), dtype=np.float32)
    pe[:, 0::2] = np.sin(pos * div)
    pe[:, 1::2] = np.cos(pos * div)
    return pe[:_S]


_PE8 = _pe8_np()

_B_SPECS = [
    ("lin0_b", 4), ("lin1_b", 4), ("lin2_b", 8), ("lin2_w", 8),
    ("gc10_b", 64), ("gc11_b", 32), ("gc20_b", 64), ("gc21_b", 32),
    ("gc30_b", 64), ("gc31_b", 32), ("gcn_ln_g", 16), ("gcn_ln_b", 16),
    ("attn_in_b", 48), ("attn_out_b", 16), ("norm1_g", 16), ("norm1_b", 16),
    ("norm2_g", 16), ("norm2_b", 16), ("enc_norm_g", 16), ("enc_norm_b", 16),
    ("pred_b", 8), ("out0_b", 4), ("out1_b", 1), ("ff2_b", 16), ("ff1_b", 2048),
]
_B_OFF = {}
_o = 0
for _n, _w in _B_SPECS:
    _B_OFF[_n] = (_o, _w)
    _o += _w
_B_LANES = _o


def _mmh_t(x, w):
    return jax.lax.dot_general(
        x, w, (((1,), (1,)), ((), ())),
        precision=jax.lax.Precision.HIGHEST,
        preferred_element_type=jnp.float32,
    )


def _mmh(x, w):
    return jax.lax.dot_general(
        x, w, (((1,), (0,)), ((), ())),
        precision=jax.lax.Precision.HIGHEST,
        preferred_element_type=jnp.float32,
    )


_BF = jnp.bfloat16


def _mmd_t(x, w):
    return jax.lax.dot_general(
        x.astype(_BF), w.astype(_BF), (((1,), (1,)), ((), ())),
        preferred_element_type=jnp.float32,
    )


def _mmd(x, w):
    return jax.lax.dot_general(
        x.astype(_BF), w.astype(_BF), (((1,), (0,)), ((), ())),
        preferred_element_type=jnp.float32,
    )


def _bf(x):
    return x.astype(_BF).astype(jnp.float32)


def _mm3(x, w):
    return _mmh(x, w)


def _ln(x, g, b, eps):
    m = jnp.mean(x, axis=-1, keepdims=True)
    v = jnp.mean((x - m) * (x - m), axis=-1, keepdims=True)
    return (x - m) / jnp.sqrt(v + eps) * g + b


def _sel(rows, cols, shift):
    r = jax.lax.broadcasted_iota(jnp.int32, (rows, cols), 0)
    c = jax.lax.broadcasted_iota(jnp.int32, (rows, cols), 1)
    return (c == r + shift).astype(jnp.float32)


def _blk(x, i):
    return x[i * _NP : (i + 1) * _NP, :]


def _fused_body(
    feat, week, stamp, a0, a1, a2, pe, bpack,
    emb1, emb2, lin0_w, lin1_w,
    gc10_w, gc11_w, gc20_w, gc21_w, gc30_w, gc31_w,
    fw0, fw1, fw2, attn_in_w, attn_out_w, ff1_w, ff2_w,
    pred_w, out0_w, out1_w,
    r1_ref, r2_ref, r2l_ref,
):
    f32 = jnp.float32

    def B(name):
        off, w = _B_OFF[name]
        return bpack[:, off : off + w]

    A0 = jnp.pad(a0[:, :], ((0, _NP - _N), (0, _NP - _N)))
    A1 = jnp.pad(a1[:, :], ((0, _NP - _N), (0, _NP - _N)))
    A2 = jnp.pad(a2[:, :], ((0, _NP - _N), (0, _NP - _N)))

    featc = jnp.pad(jnp.transpose(feat[:, :]), ((0, _NP - _N), (0, 0)))
    weekc = jnp.pad(jnp.transpose(week[:, :]), ((0, _NP - _N), (0, 0)))
    stampc = jnp.pad(jnp.transpose(stamp[:, :]), ((0, _NP - _N), (0, 0)))
    fcol = jnp.concatenate([featc[:, i : i + 1] for i in range(_S)], axis=0)
    oh_w = jnp.concatenate(
        [
            (weekc[:, i : i + 1] == jax.lax.broadcasted_iota(jnp.int32, (_NP, 8), 1)).astype(f32)
            for i in range(_S)
        ],
        axis=0,
    )
    oh_s = jnp.concatenate(
        [
            (stampc[:, i : i + 1] == jax.lax.broadcasted_iota(jnp.int32, (_NP, 5), 1)).astype(f32)
            for i in range(_S)
        ],
        axis=0,
    )

    g1 = _mmd(oh_w, emb1[:, :])
    g2 = _mmd(oh_s, emb2[:, :])
    x1e = _mmd_t(g1, lin0_w[:, :]) + B("lin0_b")
    x2e = _mmd_t(g2, lin1_w[:, :]) + B("lin1_b")
    x3e = fcol * B("lin2_w") + B("lin2_b")
    X = jnp.concatenate([x3e, x2e, x1e], axis=1)

    def gcn_branch(A, w0, b0, w1, b1):
        U = _mmd(X, w0[:, :])
        V = jnp.concatenate([_mmd(A, _blk(U, i)) for i in range(_S)], axis=0)
        H = jnp.maximum(V + b0, 0.0)
        Wd = _mmd(H, w1[:, :])
        Z = jnp.concatenate([_mmd(A, _blk(Wd, i)) for i in range(_S)], axis=0)
        return Z + b1

    z0 = gcn_branch(A0, gc10_w, B("gc10_b"), gc11_w, B("gc11_b"))
    z1 = gcn_branch(A1, gc20_w, B("gc20_b"), gc21_w, B("gc21_b"))
    z2 = gcn_branch(A2, gc30_w, B("gc30_b"), gc31_w, B("gc31_b"))
    xo = _mmd(z0, fw0[:, :]) + _mmd(z1, fw1[:, :]) + _mmd(z2, fw2[:, :])
    xg = _ln(xo + X, B("gcn_ln_g"), B("gcn_ln_b"), 1e-6)

    pe_rows = jnp.concatenate(
        [jnp.broadcast_to(pe[i : i + 1, :], (_NP, 16)) for i in range(_S)], axis=0
    )
    src = xg + pe_rows

    wq = attn_in_w[0:16, :]
    wk = attn_in_w[16:32, :]
    wv = attn_in_w[32:48, :]
    attn_in_b = B("attn_in_b")
    bq = attn_in_b[:, 0:16]
    bk = attn_in_b[:, 16:32]
    bv = attn_in_b[:, 32:48]
    q_all = _bf(_mmd_t(src, wq) + bq)
    k_all = _bf(_mmd_t(src, wk) + bk)
    v_all = _bf(_mmd_t(src, wv) + bv)

    G = (
        jax.lax.broadcasted_iota(jnp.int32, (16, 4), 0) // 4
        == jax.lax.broadcasted_iota(jnp.int32, (16, 4), 1)
    ).astype(f32)

    def tile_steps(x):
        return jnp.concatenate([x] * _S, axis=0)

    scores = []
    for j in range(_S):
        kt = tile_steps(_blk(k_all, j))
        scores.append(_mm3(q_all * kt, G) * 0.5)
    m = scores[0]
    for j in range(1, _S):
        m = jnp.maximum(m, scores[j])
    exps = [jnp.exp(s - m) for s in scores]
    den = exps[0]
    for j in range(1, _S):
        den = den + exps[j]
    ao = jnp.zeros((_R, 16), f32)
    for j in range(_S):
        vt = tile_steps(_blk(v_all, j))
        ao = ao + _mmd(exps[j] / den, G.T) * vt

    ao = _mmd_t(ao, attn_out_w[:, :]) + B("attn_out_b")
    x1 = _ln(src + ao, B("norm1_g"), B("norm1_b"), 1e-5)
    h = jnp.maximum(_mmd_t(x1, ff1_w[:, :]) + B("ff1_b"), 0.0)
    y = _mmd_t(h, ff2_w[:, :]) + B("ff2_b")
    x2 = _ln(x1 + y, B("norm2_g"), B("norm2_b"), 1e-5)
    enc = _ln(x2, B("enc_norm_g"), B("enc_norm_b"), 1e-6)

    r1 = _mmd_t(enc, pred_w[:, :]) + B("pred_b")
    rb = _mmd_t(r1, out0_w[:, :]) + B("out0_b")
    r2 = jnp.sum(_bf(rb) * _bf(out1_w[:, :]), axis=-1, keepdims=True) + B("out1_b")[0, 0]
    for i in range(_S):
        r1_ref[i] = r1[i * _NP : i * _NP + _N, :]
        r2_ref[i] = r2[i * _NP : i * _NP + _N, :]
    r2l_ref[:, :] = r2[(_S - 1) * _NP : (_S - 1) * _NP + _N, :]


def kernel(feature_tensor, week_tensor, stamptensor, a0, a1, a2, k, params):
    p = params
    del k
    bpack = jnp.concatenate(
        [p[name].reshape(1, -1) for name, _ in _B_SPECS], axis=1
    )
    return tuple(
        pl.pallas_call(
            _fused_body,
            out_shape=[
                jax.ShapeDtypeStruct((_S, _N, 8), jnp.float32),
                jax.ShapeDtypeStruct((_S, _N, 1), jnp.float32),
                jax.ShapeDtypeStruct((_N, 1), jnp.float32),
            ],
        )(
            feature_tensor, week_tensor, stamptensor, a0, a1, a2,
            jnp.asarray(_PE8), bpack,
            p["emb1"], p["emb2"], p["lin0_w"], p["lin1_w"],
            p["gc10_w"], p["gc11_w"], p["gc20_w"], p["gc21_w"],
            p["gc30_w"], p["gc31_w"],
            p["fw0"], p["fw1"], p["fw2"],
            p["attn_in_w"], p["attn_out_w"],
            p["ff1_w"], p["ff2_w"],
            p["pred_w"], p["out0_w"], p["out1_w"],
        )
    )


# speedup vs baseline: 1.2563x; 1.0588x over previous
import math

import jax
import jax.numpy as jnp
import numpy as np
from jax.experimental import pallas as pl

_S, _N, _NP = 8, 97, 104
_R = _S * _NP


def _pe8_np():
    pos = np.arange(20, dtype=np.float32)[:, None]
    div = np.exp(np.arange(0, 16, 2, dtype=np.float32) * (-math.log(10000.0) / 16.0))
    pe = np.zeros((20, 16), dtype=np.float32)
    pe[:, 0::2] = np.sin(pos * div)
    pe[:, 1::2] = np.cos(pos * div)
    return pe[:_S]


_PE8 = _pe8_np()

_B_SPECS = [
    ("lin0_b", 4), ("lin1_b", 4), ("lin2_b", 8), ("lin2_w", 8),
    ("gc10_b", 64), ("gc11_b", 32), ("gc20_b", 64), ("gc21_b", 32),
    ("gc30_b", 64), ("gc31_b", 32), ("gcn_ln_g", 16), ("gcn_ln_b", 16),
    ("attn_in_b", 48), ("attn_out_b", 16), ("norm1_g", 16), ("norm1_b", 16),
    ("norm2_g", 16), ("norm2_b", 16), ("enc_norm_g", 16), ("enc_norm_b", 16),
    ("pred_b", 8), ("out0_b", 4), ("out1_b", 1), ("ff2_b", 16), ("ff1_b", 2048),
]
_B_OFF = {}
_o = 0
for _n, _w in _B_SPECS:
    _B_OFF[_n] = (_o, _w)
    _o += _w
_B_LANES = _o


def _mmh_t(x, w):
    return jax.lax.dot_general(
        x, w, (((1,), (1,)), ((), ())),
        precision=jax.lax.Precision.HIGHEST,
        preferred_element_type=jnp.float32,
    )


def _mmh(x, w):
    return jax.lax.dot_general(
        x, w, (((1,), (0,)), ((), ())),
        precision=jax.lax.Precision.HIGHEST,
        preferred_element_type=jnp.float32,
    )


_BF = jnp.bfloat16


def _mmd_t(x, w):
    return jax.lax.dot_general(
        x.astype(_BF), w.astype(_BF), (((1,), (1,)), ((), ())),
        preferred_element_type=jnp.float32,
    )


def _mmd(x, w):
    return jax.lax.dot_general(
        x.astype(_BF), w.astype(_BF), (((1,), (0,)), ((), ())),
        preferred_element_type=jnp.float32,
    )


def _bf(x):
    return x.astype(_BF).astype(jnp.float32)


def _mm3(x, w):
    return _mmh(x, w)


def _ln(x, g, b, eps):
    m = jnp.mean(x, axis=-1, keepdims=True)
    v = jnp.mean((x - m) * (x - m), axis=-1, keepdims=True)
    return (x - m) / jnp.sqrt(v + eps) * g + b


def _sel(rows, cols, shift):
    r = jax.lax.broadcasted_iota(jnp.int32, (rows, cols), 0)
    c = jax.lax.broadcasted_iota(jnp.int32, (rows, cols), 1)
    return (c == r + shift).astype(jnp.float32)


def _blk(x, i):
    return x[i * _NP : (i + 1) * _NP, :]


def _fused_body(
    feat, week, stamp, a0, a1, a2, pe,
    emb1, emb2, lin0_w, lin1_w,
    gc10_w, gc11_w, gc20_w, gc21_w, gc30_w, gc31_w,
    fw0, fw1, fw2, attn_in_w, attn_out_w, ff1_w, ff2_w,
    pred_w, out0_w, out1_w,
    *bias_refs,
):
    f32 = jnp.float32
    r1_ref, r2_ref, r2l_ref = bias_refs[-3:]
    _bias = dict(zip([n for n, _ in _B_SPECS], bias_refs[:-3]))

    def B(name):
        return _bias[name][:, :]

    A0 = jnp.pad(a0[:, :], ((0, _NP - _N), (0, _NP - _N)))
    A1 = jnp.pad(a1[:, :], ((0, _NP - _N), (0, _NP - _N)))
    A2 = jnp.pad(a2[:, :], ((0, _NP - _N), (0, _NP - _N)))

    featc = jnp.pad(jnp.transpose(feat[:, :]), ((0, _NP - _N), (0, 0)))
    weekc = jnp.pad(jnp.transpose(week[:, :]), ((0, _NP - _N), (0, 0)))
    stampc = jnp.pad(jnp.transpose(stamp[:, :]), ((0, _NP - _N), (0, 0)))
    fcol = jnp.concatenate([featc[:, i : i + 1] for i in range(_S)], axis=0)
    oh_w = jnp.concatenate(
        [
            (weekc[:, i : i + 1] == jax.lax.broadcasted_iota(jnp.int32, (_NP, 8), 1)).astype(f32)
            for i in range(_S)
        ],
        axis=0,
    )
    oh_s = jnp.concatenate(
        [
            (stampc[:, i : i + 1] == jax.lax.broadcasted_iota(jnp.int32, (_NP, 5), 1)).astype(f32)
            for i in range(_S)
        ],
        axis=0,
    )

    g1 = _mmd(oh_w, emb1[:, :])
    g2 = _mmd(oh_s, emb2[:, :])
    x1e = _mmd_t(g1, lin0_w[:, :]) + B("lin0_b")
    x2e = _mmd_t(g2, lin1_w[:, :]) + B("lin1_b")
    x3e = fcol * B("lin2_w") + B("lin2_b")
    X = jnp.concatenate([x3e, x2e, x1e], axis=1)

    def gcn_branch(A, w0, b0, w1, b1):
        U = _mmd(X, w0[:, :])
        V = jnp.concatenate([_mmd(A, _blk(U, i)) for i in range(_S)], axis=0)
        H = jnp.maximum(V + b0, 0.0)
        Wd = _mmd(H, w1[:, :])
        Z = jnp.concatenate([_mmd(A, _blk(Wd, i)) for i in range(_S)], axis=0)
        return Z + b1

    z0 = gcn_branch(A0, gc10_w, B("gc10_b"), gc11_w, B("gc11_b"))
    z1 = gcn_branch(A1, gc20_w, B("gc20_b"), gc21_w, B("gc21_b"))
    z2 = gcn_branch(A2, gc30_w, B("gc30_b"), gc31_w, B("gc31_b"))
    xo = _mmd(z0, fw0[:, :]) + _mmd(z1, fw1[:, :]) + _mmd(z2, fw2[:, :])
    xg = _ln(xo + X, B("gcn_ln_g"), B("gcn_ln_b"), 1e-6)

    pe_rows = jnp.concatenate(
        [jnp.broadcast_to(pe[i : i + 1, :], (_NP, 16)) for i in range(_S)], axis=0
    )
    src = xg + pe_rows

    wq = attn_in_w[0:16, :]
    wk = attn_in_w[16:32, :]
    wv = attn_in_w[32:48, :]
    attn_in_b = B("attn_in_b")
    bq = attn_in_b[:, 0:16]
    bk = attn_in_b[:, 16:32]
    bv = attn_in_b[:, 32:48]
    q_all = _bf(_mmd_t(src, wq) + bq)
    k_all = _bf(_mmd_t(src, wk) + bk)
    v_all = _bf(_mmd_t(src, wv) + bv)

    G = (
        jax.lax.broadcasted_iota(jnp.int32, (16, 4), 0) // 4
        == jax.lax.broadcasted_iota(jnp.int32, (16, 4), 1)
    ).astype(f32)

    def tile_steps(x):
        return jnp.concatenate([x] * _S, axis=0)

    scores = []
    for j in range(_S):
        kt = tile_steps(_blk(k_all, j))
        scores.append(_mm3(q_all * kt, G) * 0.5)
    m = scores[0]
    for j in range(1, _S):
        m = jnp.maximum(m, scores[j])
    exps = [jnp.exp(s - m) for s in scores]
    den = exps[0]
    for j in range(1, _S):
        den = den + exps[j]
    ao = jnp.zeros((_R, 16), f32)
    for j in range(_S):
        vt = tile_steps(_blk(v_all, j))
        ao = ao + _mmd(exps[j] / den, G.T) * vt

    ao = _mmd_t(ao, attn_out_w[:, :]) + B("attn_out_b")
    x1 = _ln(src + ao, B("norm1_g"), B("norm1_b"), 1e-5)
    h = jnp.maximum(_mmd_t(x1, ff1_w[:, :]) + B("ff1_b"), 0.0)
    y = _mmd_t(h, ff2_w[:, :]) + B("ff2_b")
    x2 = _ln(x1 + y, B("norm2_g"), B("norm2_b"), 1e-5)
    enc = _ln(x2, B("enc_norm_g"), B("enc_norm_b"), 1e-6)

    r1 = _mmd_t(enc, pred_w[:, :]) + B("pred_b")
    rb = _mmd_t(r1, out0_w[:, :]) + B("out0_b")
    r2 = jnp.sum(_bf(rb) * _bf(out1_w[:, :]), axis=-1, keepdims=True) + B("out1_b")[0, 0]
    for i in range(_S):
        r1_ref[i] = r1[i * _NP : i * _NP + _N, :]
        r2_ref[i] = r2[i * _NP : i * _NP + _N, :]
    r2l_ref[:, :] = r2[(_S - 1) * _NP : (_S - 1) * _NP + _N, :]


def kernel(feature_tensor, week_tensor, stamptensor, a0, a1, a2, k, params):
    p = params
    del k
    return tuple(
        pl.pallas_call(
            _fused_body,
            out_shape=[
                jax.ShapeDtypeStruct((_S, _N, 8), jnp.float32),
                jax.ShapeDtypeStruct((_S, _N, 1), jnp.float32),
                jax.ShapeDtypeStruct((_N, 1), jnp.float32),
            ],
        )(
            feature_tensor, week_tensor, stamptensor, a0, a1, a2,
            jnp.asarray(_PE8),
            p["emb1"], p["emb2"], p["lin0_w"], p["lin1_w"],
            p["gc10_w"], p["gc11_w"], p["gc20_w"], p["gc21_w"],
            p["gc30_w"], p["gc31_w"],
            p["fw0"], p["fw1"], p["fw2"],
            p["attn_in_w"], p["attn_out_w"],
            p["ff1_w"], p["ff2_w"],
            p["pred_w"], p["out0_w"], p["out1_w"],
            *[p[name].reshape(1, -1) for name, _ in _B_SPECS],
        )
    )
